# Initial kernel scaffold; baseline (speedup 1.0000x reference)
import jax
import jax.numpy as jnp
from jax.experimental import pallas as pl


def kernel(x, edge_index, batch, W1, b1, W2, b2):
    raise NotImplementedError("write your pallas kernel here")


# speedup vs baseline: 9.2813x; 9.2813x over previous
import functools

import jax
import jax.numpy as jnp
from jax import lax
from jax.experimental import pallas as pl
from jax.experimental.pallas import tpu as pltpu
from jax.experimental.pallas import tpu_sc as plsc

N = 10000
E = 320000
DIN = 128
DOUT = 128
G = 16

NC = 2
NS = 16
LANES = 16
K = 80
ZR = 208

BN = 1000
NB = N // BN

@functools.lru_cache(maxsize=None)
def _mesh():
    return plsc.VectorSubcoreMesh(
        core_axis_name="c", subcore_axis_name="s",
        num_cores=NC, num_subcores=NS)

_f32 = jnp.float32


def _zero_shared(zbuf, shared, s, width):
    @pl.loop(0, ZR)
    def _(i):
        @pl.loop(0, width // LANES)
        def _(j):
            zbuf[i, pl.ds(j * LANES, LANES)] = jnp.zeros((LANES,), _f32)

    @pl.loop(0, 3)
    def _(d):
        pltpu.sync_copy(zbuf, shared.at[pl.ds(s * 624 + d * ZR, ZR)])

    @pl.when(s == NS - 1)
    def _():
        pltpu.sync_copy(zbuf.at[pl.ds(0, 16)], shared.at[pl.ds(9984, 16)])


DEGW = 128


def _sc_degree(dst):
    epw = E // (NC * NS)

    @functools.partial(
        pl.kernel,
        out_type=jax.ShapeDtypeStruct((NC * N, DEGW), _f32),
        mesh=_mesh(),
        scratch_types=[
            pltpu.VMEM((K,), jnp.int32),
            pltpu.VMEM((K, DEGW), _f32),
            pltpu.VMEM((ZR, DEGW), _f32),
            pltpu.VMEM_SHARED((N, DEGW), _f32),
        ],
    )
    def deg_kernel(dst_hbm, out_hbm, idx_v, ones_v, zbuf, shared):
        c = lax.axis_index("c")
        s = lax.axis_index("s")
        _zero_shared(zbuf, shared, s, DEGW)

        @pl.loop(0, K)
        def _(i):
            @pl.loop(0, DEGW // LANES)
            def _(j):
                ones_v[i, pl.ds(j * LANES, LANES)] = jnp.ones((LANES,), _f32)

        plsc.subcore_barrier()

        base = (c * NS + s) * epw

        @pl.loop(0, epw // K)
        def _(t):
            pltpu.sync_copy(dst_hbm.at[pl.ds(base + t * K, K)], idx_v)
            pltpu.sync_copy(ones_v, shared.at[idx_v], add=True)

        plsc.subcore_barrier()

        @pl.when(s == 0)
        def _():
            pltpu.sync_copy(shared, out_hbm.at[pl.ds(c * N, N)])

    return deg_kernel(dst)


@functools.lru_cache(maxsize=None)
def _make_agg(width, col_split):
    epw = E // NS if col_split else E // (NC * NS)

    @functools.partial(
        pl.kernel,
        out_type=jax.ShapeDtypeStruct((NC * N, width), _f32),
        mesh=_mesh(),
        scratch_types=[
            pltpu.VMEM((K,), jnp.int32),
            pltpu.VMEM((K,), jnp.int32),
            pltpu.VMEM((K, width), _f32),
            pltpu.VMEM((ZR, width), _f32),
            pltpu.VMEM_SHARED((N, width), _f32),
        ],
    )
    def agg_kernel(g_hbm, src_hbm, dst_hbm, out_hbm,
                   sidx, didx, rows, zbuf, shared):
        c = lax.axis_index("c")
        s = lax.axis_index("s")
        _zero_shared(zbuf, shared, s, width)
        plsc.subcore_barrier()

        if col_split:
            base = s * epw
            off = c * N
        else:
            base = (c * NS + s) * epw

        @pl.loop(0, epw // K)
        def _(t):
            b = base + t * K
            pltpu.sync_copy(src_hbm.at[pl.ds(b, K)], sidx)
            pltpu.sync_copy(dst_hbm.at[pl.ds(b, K)], didx)
            if col_split:
                @pl.loop(0, K // LANES)
                def _(j):
                    sidx[pl.ds(j * LANES, LANES)] = (
                        sidx[pl.ds(j * LANES, LANES)] + off)
            pltpu.sync_copy(g_hbm.at[sidx], rows)
            pltpu.sync_copy(rows, shared.at[didx], add=True)

        plsc.subcore_barrier()

        @pl.when(s == 0)
        def _():
            pltpu.sync_copy(shared, out_hbm.at[pl.ds(c * N, N)])

    return agg_kernel


def _tc_prep_body(x_ref, w_ref, dega_ref, degb_ref, out_ref):
    x = x_ref[...]
    xn = x / jnp.clip(jnp.sum(x, axis=-1, keepdims=True), 1.0, None)
    deg = dega_ref[:, :1] + degb_ref[:, :1] + 1.0
    dinv = lax.rsqrt(deg)
    h = lax.dot_general(xn, w_ref[...], (((1,), (0,)), ((), ())),
                        preferred_element_type=_f32,
                        precision=lax.Precision.HIGHEST)
    out_ref[0, :, :] = dinv * h


def _tc_prep(x, W1, degp):
    return pl.pallas_call(
        _tc_prep_body,
        grid=(NB, 2),
        in_specs=[
            pl.BlockSpec((BN, DIN), lambda i, c: (i, 0)),
            pl.BlockSpec((DIN, DOUT), lambda i, c: (0, c)),
            pl.BlockSpec((BN, DEGW), lambda i, c: (i, 0)),
            pl.BlockSpec((BN, DEGW), lambda i, c: (i + NB, 0)),
        ],
        out_specs=pl.BlockSpec((1, BN, DOUT), lambda i, c: (c, i, 0)),
        out_shape=jax.ShapeDtypeStruct((2, N, DOUT), _f32),
    )(x, W1, degp, degp)


def _tc_mid_body(agga_ref, aggb_ref, g1a_ref, g1b_ref, dega_ref, degb_ref,
                 b1_ref, w2a_ref, w2b_ref, out_ref):
    deg = dega_ref[:, :1] + degb_ref[:, :1] + 1.0
    dinv = lax.rsqrt(deg)
    b1 = b1_ref[...]
    z1a = jnp.maximum(dinv * (agga_ref[...] + g1a_ref[...]) + b1[0:1, :], 0.0)
    z1b = jnp.maximum(dinv * (aggb_ref[...] + g1b_ref[...]) + b1[1:2, :], 0.0)
    dn = (((1,), (0,)), ((), ()))
    h2 = (lax.dot_general(z1a, w2a_ref[0], dn, preferred_element_type=_f32,
                          precision=lax.Precision.HIGHEST)
          + lax.dot_general(z1b, w2b_ref[0], dn, preferred_element_type=_f32,
                            precision=lax.Precision.HIGHEST))
    out_ref[...] = dinv * h2


def _tc_mid(agg1, g1f, degp, b1, W2):
    w2r = W2.reshape(2, DOUT, DOUT)
    b1r = b1.reshape(2, DOUT)
    return pl.pallas_call(
        _tc_mid_body,
        grid=(NB,),
        in_specs=[
            pl.BlockSpec((BN, DOUT), lambda i: (i, 0)),
            pl.BlockSpec((BN, DOUT), lambda i: (i + NB, 0)),
            pl.BlockSpec((BN, DOUT), lambda i: (i, 0)),
            pl.BlockSpec((BN, DOUT), lambda i: (i + NB, 0)),
            pl.BlockSpec((BN, DEGW), lambda i: (i, 0)),
            pl.BlockSpec((BN, DEGW), lambda i: (i + NB, 0)),
            pl.BlockSpec((2, DOUT), lambda i: (0, 0)),
            pl.BlockSpec((1, DOUT, DOUT), lambda i: (0, 0, 0)),
            pl.BlockSpec((1, DOUT, DOUT), lambda i: (1, 0, 0)),
        ],
        out_specs=pl.BlockSpec((BN, DOUT), lambda i: (i, 0)),
        out_shape=jax.ShapeDtypeStruct((N, DOUT), _f32),
    )(agg1, agg1, g1f, g1f, degp, degp, b1r, w2r, w2r)


def _tc_final_body(agga_ref, aggb_ref, g2_ref, dega_ref, degb_ref,
                   b2_ref, bat_ref, out_ref, s_acc, c_acc):
    i = pl.program_id(0)

    @pl.when(i == 0)
    def _():
        s_acc[...] = jnp.zeros((G, DOUT), _f32)
        c_acc[...] = jnp.zeros((G, DOUT), _f32)

    deg = dega_ref[:, :1] + degb_ref[:, :1] + 1.0
    dinv = lax.rsqrt(deg)
    z2 = jnp.maximum(
        dinv * (agga_ref[...] + aggb_ref[...] + g2_ref[...]) + b2_ref[...],
        0.0)
    gids = lax.broadcasted_iota(jnp.int32, (BN, G), 1).astype(_f32)
    onehot = jnp.where(bat_ref[...] == gids, 1.0, 0.0)
    dn = (((0,), (0,)), ((), ()))
    s_acc[...] += lax.dot_general(onehot, z2, dn,
                                  preferred_element_type=_f32,
                                  precision=lax.Precision.HIGHEST)
    c_acc[...] += jnp.sum(onehot, axis=0)[:, None]

    @pl.when(i == pl.num_programs(0) - 1)
    def _():
        out_ref[...] = s_acc[...] / jnp.clip(c_acc[...], 1.0, None)


def _tc_final(agg2, g2, degp, b2, batchf):
    return pl.pallas_call(
        _tc_final_body,
        grid=(NB,),
        in_specs=[
            pl.BlockSpec((BN, DOUT), lambda i: (i, 0)),
            pl.BlockSpec((BN, DOUT), lambda i: (i + NB, 0)),
            pl.BlockSpec((BN, DOUT), lambda i: (i, 0)),
            pl.BlockSpec((BN, DEGW), lambda i: (i, 0)),
            pl.BlockSpec((BN, DEGW), lambda i: (i + NB, 0)),
            pl.BlockSpec((1, DOUT), lambda i: (0, 0)),
            pl.BlockSpec((BN, 1), lambda i: (i, 0)),
        ],
        out_specs=pl.BlockSpec((G, DOUT), lambda i: (0, 0)),
        out_shape=jax.ShapeDtypeStruct((G, DOUT), _f32),
        scratch_shapes=[pltpu.VMEM((G, DOUT), _f32),
                        pltpu.VMEM((G, DOUT), _f32)],
    )(agg2, agg2, g2, degp, degp, b2.reshape(1, DOUT), batchf)


def _agg_l1(g1f, src, dst):
    return _make_agg(DOUT, True)(g1f, src, dst)


def _agg_l2(g2, src, dst):
    return _make_agg(DOUT, False)(g2, src, dst)


def kernel(x, edge_index, batch, W1, b1, W2, b2):
    src = edge_index[0]
    dst = edge_index[1]
    batchf = batch.astype(_f32).reshape(N, 1)

    degp = _sc_degree(dst)
    g1 = _tc_prep(x, W1, degp)
    g1f = g1.reshape(2 * N, DOUT)
    agg1 = _agg_l1(g1f, src, dst)
    g2 = _tc_mid(agg1, g1f, degp, b1, W2)
    agg2 = _agg_l2(g2, src, dst)
    return _tc_final(agg2, g2, degp, b2, batchf)


# speedup vs baseline: 15.5438x; 1.6747x over previous
import functools

import jax
import jax.numpy as jnp
from jax import lax
from jax.experimental import pallas as pl
from jax.experimental.pallas import tpu as pltpu
from jax.experimental.pallas import tpu_sc as plsc

N = 10000
E = 320000
DIN = 128
DOUT = 128
G = 16

NC = 2
NS = 16
LANES = 16
K = 80
ZR = 48
NBLK = 5

BN = 1000
NB = N // BN

@functools.lru_cache(maxsize=None)
def _mesh():
    return plsc.VectorSubcoreMesh(
        core_axis_name="c", subcore_axis_name="s",
        num_cores=NC, num_subcores=NS)

_f32 = jnp.float32


def _zero_shared(zbuf, shared, s, width):
    @pl.loop(0, ZR)
    def _(i):
        @pl.loop(0, width // LANES)
        def _(j):
            zbuf[i, pl.ds(j * LANES, LANES)] = jnp.zeros((LANES,), _f32)

    @pl.loop(0, 13)
    def _(d):
        pltpu.sync_copy(zbuf, shared.at[pl.ds(s * 624 + d * ZR, ZR)])

    @pl.when(s == NS - 1)
    def _():
        pltpu.sync_copy(zbuf.at[pl.ds(0, 16)], shared.at[pl.ds(9984, 16)])


DEGW = 128


def _sc_degree(dst):
    epw = E // (NC * NS)

    @functools.partial(
        pl.kernel,
        out_type=jax.ShapeDtypeStruct((NC * N, DEGW), _f32),
        mesh=_mesh(),
        scratch_types=[
            pltpu.VMEM((K,), jnp.int32),
            pltpu.VMEM((K, DEGW), _f32),
            pltpu.VMEM((ZR, DEGW), _f32),
            pltpu.VMEM_SHARED((N, DEGW), _f32),
        ],
    )
    def deg_kernel(dst_hbm, out_hbm, idx_v, ones_v, zbuf, shared):
        c = lax.axis_index("c")
        s = lax.axis_index("s")
        _zero_shared(zbuf, shared, s, DEGW)

        @pl.loop(0, K)
        def _(i):
            @pl.loop(0, DEGW // LANES)
            def _(j):
                ones_v[i, pl.ds(j * LANES, LANES)] = jnp.ones((LANES,), _f32)

        plsc.subcore_barrier()

        base = (c * NS + s) * epw

        @pl.loop(0, epw // K)
        def _(t):
            pltpu.sync_copy(dst_hbm.at[pl.ds(base + t * K, K)], idx_v)
            pltpu.sync_copy(ones_v, shared.at[idx_v], add=True)

        plsc.subcore_barrier()

        @pl.when(s == 0)
        def _():
            pltpu.sync_copy(shared, out_hbm.at[pl.ds(c * N, N)])

    return deg_kernel(dst)


@functools.lru_cache(maxsize=None)
def _make_agg(width, col_split):
    epw = E // NS if col_split else E // (NC * NS)
    TB = epw // K // NBLK

    @functools.partial(
        pl.kernel,
        out_type=jax.ShapeDtypeStruct((NC * N, width), _f32),
        mesh=_mesh(),
        scratch_types=[
            pltpu.VMEM((TB, K), jnp.int32),
            pltpu.VMEM((TB, K), jnp.int32),
            pltpu.VMEM((K, width), _f32),
            pltpu.VMEM((K, width), _f32),
            pltpu.VMEM((ZR, width), _f32),
            pltpu.VMEM_SHARED((N, width), _f32),
            pltpu.SemaphoreType.DMA,
            pltpu.SemaphoreType.DMA,
            pltpu.SemaphoreType.DMA,
            pltpu.SemaphoreType.DMA,
        ],
    )
    def agg_kernel(g_hbm, src_hbm, dst_hbm, out_hbm,
                   sidx, didx, rows0, rows1, zbuf, shared,
                   gsem0, gsem1, ssem0, ssem1):
        c = lax.axis_index("c")
        s = lax.axis_index("s")
        _zero_shared(zbuf, shared, s, width)

        w = s if col_split else c * NS + s

        def gather(t, rows, sem):
            pltpu.async_copy(g_hbm.at[sidx.at[t]], rows, sem)

        def gather_wait(t, rows, sem):
            pltpu.make_async_copy(g_hbm.at[sidx.at[t]], rows, sem).wait()

        def scat(t, rows, sem):
            pltpu.async_copy(rows, shared.at[didx.at[t]], sem, add=True)

        def scat_wait(t, rows, sem):
            pltpu.make_async_copy(rows, shared.at[didx.at[t]], sem).wait()

        plsc.subcore_barrier()

        @pl.loop(0, NBLK)
        def _(blk):
            pltpu.sync_copy(src_hbm.at[w, blk], sidx)
            pltpu.sync_copy(dst_hbm.at[w, blk], didx)
            if col_split:
                off = c * N

                @pl.loop(0, TB)
                def _(t):
                    @pl.loop(0, K // LANES)
                    def _(j):
                        sidx[t, pl.ds(j * LANES, LANES)] = (
                            sidx[t, pl.ds(j * LANES, LANES)] + off)

            @pl.loop(0, TB // 2)
            def _(u):
                a, b = 2 * u, 2 * u + 1

                @pl.when(u > 0)
                def _():
                    scat_wait(a, rows0, ssem0)
                gather(a, rows0, gsem0)

                @pl.when(u > 0)
                def _():
                    scat_wait(b, rows1, ssem1)
                gather(b, rows1, gsem1)

                gather_wait(a, rows0, gsem0)
                scat(a, rows0, ssem0)
                gather_wait(b, rows1, gsem1)
                scat(b, rows1, ssem1)

            scat_wait(0, rows0, ssem0)
            scat_wait(1, rows1, ssem1)

            if TB % 2 == 1:
                t = TB - 1
                gather(t, rows0, gsem0)
                gather_wait(t, rows0, gsem0)
                scat(t, rows0, ssem0)
                scat_wait(t, rows0, ssem0)

        plsc.subcore_barrier()

        @pl.when(s == 0)
        def _():
            pltpu.sync_copy(shared, out_hbm.at[pl.ds(c * N, N)])

    return agg_kernel


def _tc_prep_body(x_ref, w_ref, dega_ref, degb_ref, out_ref):
    x = x_ref[...]
    xn = x / jnp.clip(jnp.sum(x, axis=-1, keepdims=True), 1.0, None)
    deg = dega_ref[:, :1] + degb_ref[:, :1] + 1.0
    dinv = lax.rsqrt(deg)
    h = lax.dot_general(xn, w_ref[...], (((1,), (0,)), ((), ())),
                        preferred_element_type=_f32,
                        precision=lax.Precision.HIGHEST)
    out_ref[0, :, :] = dinv * h


def _tc_prep(x, W1, degp):
    return pl.pallas_call(
        _tc_prep_body,
        grid=(NB, 2),
        in_specs=[
            pl.BlockSpec((BN, DIN), lambda i, c: (i, 0)),
            pl.BlockSpec((DIN, DOUT), lambda i, c: (0, c)),
            pl.BlockSpec((BN, DEGW), lambda i, c: (i, 0)),
            pl.BlockSpec((BN, DEGW), lambda i, c: (i + NB, 0)),
        ],
        out_specs=pl.BlockSpec((1, BN, DOUT), lambda i, c: (c, i, 0)),
        out_shape=jax.ShapeDtypeStruct((2, N, DOUT), _f32),
    )(x, W1, degp, degp)


def _tc_mid_body(agga_ref, aggb_ref, g1a_ref, g1b_ref, dega_ref, degb_ref,
                 b1_ref, w2a_ref, w2b_ref, out_ref):
    deg = dega_ref[:, :1] + degb_ref[:, :1] + 1.0
    dinv = lax.rsqrt(deg)
    b1 = b1_ref[...]
    z1a = jnp.maximum(dinv * (agga_ref[...] + g1a_ref[...]) + b1[0:1, :], 0.0)
    z1b = jnp.maximum(dinv * (aggb_ref[...] + g1b_ref[...]) + b1[1:2, :], 0.0)
    dn = (((1,), (0,)), ((), ()))
    h2 = (lax.dot_general(z1a, w2a_ref[0], dn, preferred_element_type=_f32,
                          precision=lax.Precision.HIGHEST)
          + lax.dot_general(z1b, w2b_ref[0], dn, preferred_element_type=_f32,
                            precision=lax.Precision.HIGHEST))
    out_ref[...] = dinv * h2


def _tc_mid(agg1, g1f, degp, b1, W2):
    w2r = W2.reshape(2, DOUT, DOUT)
    b1r = b1.reshape(2, DOUT)
    return pl.pallas_call(
        _tc_mid_body,
        grid=(NB,),
        in_specs=[
            pl.BlockSpec((BN, DOUT), lambda i: (i, 0)),
            pl.BlockSpec((BN, DOUT), lambda i: (i + NB, 0)),
            pl.BlockSpec((BN, DOUT), lambda i: (i, 0)),
            pl.BlockSpec((BN, DOUT), lambda i: (i + NB, 0)),
            pl.BlockSpec((BN, DEGW), lambda i: (i, 0)),
            pl.BlockSpec((BN, DEGW), lambda i: (i + NB, 0)),
            pl.BlockSpec((2, DOUT), lambda i: (0, 0)),
            pl.BlockSpec((1, DOUT, DOUT), lambda i: (0, 0, 0)),
            pl.BlockSpec((1, DOUT, DOUT), lambda i: (1, 0, 0)),
        ],
        out_specs=pl.BlockSpec((BN, DOUT), lambda i: (i, 0)),
        out_shape=jax.ShapeDtypeStruct((N, DOUT), _f32),
    )(agg1, agg1, g1f, g1f, degp, degp, b1r, w2r, w2r)


def _tc_final_body(agga_ref, aggb_ref, g2_ref, dega_ref, degb_ref,
                   b2_ref, bat_ref, out_ref, s_acc, c_acc):
    i = pl.program_id(0)

    @pl.when(i == 0)
    def _():
        s_acc[...] = jnp.zeros((G, DOUT), _f32)
        c_acc[...] = jnp.zeros((G, DOUT), _f32)

    deg = dega_ref[:, :1] + degb_ref[:, :1] + 1.0
    dinv = lax.rsqrt(deg)
    z2 = jnp.maximum(
        dinv * (agga_ref[...] + aggb_ref[...] + g2_ref[...]) + b2_ref[...],
        0.0)
    gids = lax.broadcasted_iota(jnp.int32, (BN, G), 1).astype(_f32)
    onehot = jnp.where(bat_ref[...] == gids, 1.0, 0.0)
    dn = (((0,), (0,)), ((), ()))
    s_acc[...] += lax.dot_general(onehot, z2, dn,
                                  preferred_element_type=_f32,
                                  precision=lax.Precision.HIGHEST)
    c_acc[...] += jnp.sum(onehot, axis=0)[:, None]

    @pl.when(i == pl.num_programs(0) - 1)
    def _():
        out_ref[...] = s_acc[...] / jnp.clip(c_acc[...], 1.0, None)


def _tc_final(agg2, g2, degp, b2, batchf):
    return pl.pallas_call(
        _tc_final_body,
        grid=(NB,),
        in_specs=[
            pl.BlockSpec((BN, DOUT), lambda i: (i, 0)),
            pl.BlockSpec((BN, DOUT), lambda i: (i + NB, 0)),
            pl.BlockSpec((BN, DOUT), lambda i: (i, 0)),
            pl.BlockSpec((BN, DEGW), lambda i: (i, 0)),
            pl.BlockSpec((BN, DEGW), lambda i: (i + NB, 0)),
            pl.BlockSpec((1, DOUT), lambda i: (0, 0)),
            pl.BlockSpec((BN, 1), lambda i: (i, 0)),
        ],
        out_specs=pl.BlockSpec((G, DOUT), lambda i: (0, 0)),
        out_shape=jax.ShapeDtypeStruct((G, DOUT), _f32),
        scratch_shapes=[pltpu.VMEM((G, DOUT), _f32),
                        pltpu.VMEM((G, DOUT), _f32)],
    )(agg2, agg2, g2, degp, degp, b2.reshape(1, DOUT), batchf)


def _agg_l1(g1f, src, dst):
    shp = (NS, NBLK, E // NS // K // NBLK, K)
    return _make_agg(DOUT, True)(g1f, src.reshape(shp), dst.reshape(shp))


def _agg_l2(g2, src, dst):
    shp = (NC * NS, NBLK, E // (NC * NS) // K // NBLK, K)
    return _make_agg(DOUT, False)(g2, src.reshape(shp), dst.reshape(shp))


def kernel(x, edge_index, batch, W1, b1, W2, b2):
    src = edge_index[0]
    dst = edge_index[1]
    batchf = batch.astype(_f32).reshape(N, 1)

    degp = _sc_degree(dst)
    g1 = _tc_prep(x, W1, degp)
    g1f = g1.reshape(2 * N, DOUT)
    agg1 = _agg_l1(g1f, src, dst)
    g2 = _tc_mid(agg1, g1f, degp, b1, W2)
    agg2 = _agg_l2(g2, src, dst)
    return _tc_final(agg2, g2, degp, b2, batchf)


# speedup vs baseline: 18.3219x; 1.1787x over previous
import functools

import jax
import jax.numpy as jnp
from jax import lax
from jax.experimental import pallas as pl
from jax.experimental.pallas import tpu as pltpu
from jax.experimental.pallas import tpu_sc as plsc

N = 10000
E = 320000
DIN = 128
DOUT = 128
G = 16

NC = 2
NS = 16
LANES = 16
K = 80
ZR = 48
NBLK = 5

BN = 1000
NB = N // BN

@functools.lru_cache(maxsize=None)
def _mesh():
    return plsc.VectorSubcoreMesh(
        core_axis_name="c", subcore_axis_name="s",
        num_cores=NC, num_subcores=NS)

_f32 = jnp.float32


def _zero_shared(zbuf, shared, s, width):
    @pl.loop(0, ZR)
    def _(i):
        @pl.loop(0, width // LANES)
        def _(j):
            zbuf[i, pl.ds(j * LANES, LANES)] = jnp.zeros((LANES,), _f32)

    @pl.loop(0, 13)
    def _(d):
        pltpu.sync_copy(zbuf, shared.at[pl.ds(s * 624 + d * ZR, ZR)])

    @pl.when(s == NS - 1)
    def _():
        pltpu.sync_copy(zbuf.at[pl.ds(0, 16)], shared.at[pl.ds(9984, 16)])


HR = 80


def _sc_degree(dst):
    epw = E // (NC * NS)
    import dataclasses

    @functools.partial(
        pl.kernel,
        out_type=jax.ShapeDtypeStruct((NC * HR, 128), _f32),
        mesh=_mesh(),
        scratch_types=[
            pltpu.VMEM((epw,), jnp.int32),
            pltpu.VMEM((HR, 128), _f32),
            pltpu.VMEM((HR,), jnp.int32),
            pltpu.VMEM_SHARED((HR, 128), _f32),
        ],
        compiler_params=dataclasses.replace(pltpu.CompilerParams(),
                                            needs_layout_passes=False),
    )
    def deg_kernel(dst_hbm, out_hbm, idx_v, hist, iotab, shared):
        c = lax.axis_index("c")
        s = lax.axis_index("s")

        @pl.loop(0, HR)
        def _(i):
            @pl.loop(0, 8)
            def _(j):
                hist[i, pl.ds(j * LANES, LANES)] = jnp.zeros((LANES,), _f32)

        @pl.loop(0, HR // LANES)
        def _(i):
            iotab[pl.ds(i * LANES, LANES)] = (
                lax.iota(jnp.int32, LANES) + i * LANES)

        plsc.subcore_barrier()

        @pl.when(s == 0)
        def _():
            pltpu.sync_copy(hist, shared)

        pltpu.sync_copy(dst_hbm.at[c * NS + s], idx_v)
        plsc.subcore_barrier()

        ones = jnp.ones((LANES,), _f32)

        @pl.loop(0, epw // LANES)
        def _(t):
            iv = idx_v[pl.ds(t * LANES, LANES)]
            row = lax.shift_right_logical(iv, 7)
            lane = lax.bitwise_and(iv, 127)
            plsc.addupdate_scatter(hist, [row, lane], ones)

        plsc.subcore_barrier()
        pltpu.sync_copy(hist, shared.at[iotab], add=True)
        plsc.subcore_barrier()

        @pl.when(s == 0)
        def _():
            pltpu.sync_copy(shared, out_hbm.at[pl.ds(c * HR, HR)])

    return deg_kernel(dst.reshape(NC * NS, epw))


@functools.lru_cache(maxsize=None)
def _make_agg(width, col_split):
    epw = E // NS if col_split else E // (NC * NS)
    TB = epw // K // NBLK

    @functools.partial(
        pl.kernel,
        out_type=jax.ShapeDtypeStruct((NC * N, width), _f32),
        mesh=_mesh(),
        scratch_types=[
            pltpu.VMEM((TB, K), jnp.int32),
            pltpu.VMEM((TB, K), jnp.int32),
            pltpu.VMEM((K, width), _f32),
            pltpu.VMEM((K, width), _f32),
            pltpu.VMEM((ZR, width), _f32),
            pltpu.VMEM_SHARED((N, width), _f32),
            pltpu.SemaphoreType.DMA,
            pltpu.SemaphoreType.DMA,
            pltpu.SemaphoreType.DMA,
            pltpu.SemaphoreType.DMA,
        ],
    )
    def agg_kernel(g_hbm, src_hbm, dst_hbm, out_hbm,
                   sidx, didx, rows0, rows1, zbuf, shared,
                   gsem0, gsem1, ssem0, ssem1):
        c = lax.axis_index("c")
        s = lax.axis_index("s")
        _zero_shared(zbuf, shared, s, width)

        w = s if col_split else c * NS + s

        def gather(t, rows, sem):
            pltpu.async_copy(g_hbm.at[sidx.at[t]], rows, sem)

        def gather_wait(t, rows, sem):
            pltpu.make_async_copy(g_hbm.at[sidx.at[t]], rows, sem).wait()

        def scat(t, rows, sem):
            pltpu.async_copy(rows, shared.at[didx.at[t]], sem, add=True)

        def scat_wait(t, rows, sem):
            pltpu.make_async_copy(rows, shared.at[didx.at[t]], sem).wait()

        plsc.subcore_barrier()

        @pl.loop(0, NBLK)
        def _(blk):
            pltpu.sync_copy(src_hbm.at[w, blk], sidx)
            pltpu.sync_copy(dst_hbm.at[w, blk], didx)
            if col_split:
                off = c * N

                @pl.loop(0, TB)
                def _(t):
                    @pl.loop(0, K // LANES)
                    def _(j):
                        sidx[t, pl.ds(j * LANES, LANES)] = (
                            sidx[t, pl.ds(j * LANES, LANES)] + off)

            @pl.loop(0, TB // 2)
            def _(u):
                a, b = 2 * u, 2 * u + 1

                @pl.when(u > 0)
                def _():
                    scat_wait(a, rows0, ssem0)
                gather(a, rows0, gsem0)

                @pl.when(u > 0)
                def _():
                    scat_wait(b, rows1, ssem1)
                gather(b, rows1, gsem1)

                gather_wait(a, rows0, gsem0)
                scat(a, rows0, ssem0)
                gather_wait(b, rows1, gsem1)
                scat(b, rows1, ssem1)

            scat_wait(0, rows0, ssem0)
            scat_wait(1, rows1, ssem1)

            if TB % 2 == 1:
                t = TB - 1
                gather(t, rows0, gsem0)
                gather_wait(t, rows0, gsem0)
                scat(t, rows0, ssem0)
                scat_wait(t, rows0, ssem0)

        plsc.subcore_barrier()

        @pl.when(s == 0)
        def _():
            pltpu.sync_copy(shared, out_hbm.at[pl.ds(c * N, N)])

    return agg_kernel


def _tc_prep_body(x_ref, w_ref, dega_ref, degb_ref, out_ref):
    x = x_ref[...]
    xn = x / jnp.clip(jnp.sum(x, axis=-1, keepdims=True), 1.0, None)
    deg = dega_ref[...] + degb_ref[...] + 1.0
    dinv = lax.rsqrt(deg)
    h = lax.dot_general(xn, w_ref[...], (((1,), (0,)), ((), ())),
                        preferred_element_type=_f32,
                        precision=lax.Precision.HIGHEST)
    out_ref[0, :, :] = dinv * h


def _tc_prep(x, W1, dega, degb):
    return pl.pallas_call(
        _tc_prep_body,
        grid=(NB, 2),
        in_specs=[
            pl.BlockSpec((BN, DIN), lambda i, c: (i, 0)),
            pl.BlockSpec((DIN, DOUT), lambda i, c: (0, c)),
            pl.BlockSpec((BN, 1), lambda i, c: (i, 0)),
            pl.BlockSpec((BN, 1), lambda i, c: (i, 0)),
        ],
        out_specs=pl.BlockSpec((1, BN, DOUT), lambda i, c: (c, i, 0)),
        out_shape=jax.ShapeDtypeStruct((2, N, DOUT), _f32),
    )(x, W1, dega, degb)


def _tc_mid_body(agga_ref, aggb_ref, g1a_ref, g1b_ref, dega_ref, degb_ref,
                 b1_ref, w2a_ref, w2b_ref, out_ref):
    deg = dega_ref[...] + degb_ref[...] + 1.0
    dinv = lax.rsqrt(deg)
    b1 = b1_ref[...]
    z1a = jnp.maximum(dinv * (agga_ref[...] + g1a_ref[...]) + b1[0:1, :], 0.0)
    z1b = jnp.maximum(dinv * (aggb_ref[...] + g1b_ref[...]) + b1[1:2, :], 0.0)
    dn = (((1,), (0,)), ((), ()))
    h2 = (lax.dot_general(z1a, w2a_ref[0], dn, preferred_element_type=_f32,
                          precision=lax.Precision.HIGHEST)
          + lax.dot_general(z1b, w2b_ref[0], dn, preferred_element_type=_f32,
                            precision=lax.Precision.HIGHEST))
    out_ref[...] = dinv * h2


def _tc_mid(agg1, g1f, dega, degb, b1, W2):
    w2r = W2.reshape(2, DOUT, DOUT)
    b1r = b1.reshape(2, DOUT)
    return pl.pallas_call(
        _tc_mid_body,
        grid=(NB,),
        in_specs=[
            pl.BlockSpec((BN, DOUT), lambda i: (i, 0)),
            pl.BlockSpec((BN, DOUT), lambda i: (i + NB, 0)),
            pl.BlockSpec((BN, DOUT), lambda i: (i, 0)),
            pl.BlockSpec((BN, DOUT), lambda i: (i + NB, 0)),
            pl.BlockSpec((BN, 1), lambda i: (i, 0)),
            pl.BlockSpec((BN, 1), lambda i: (i, 0)),
            pl.BlockSpec((2, DOUT), lambda i: (0, 0)),
            pl.BlockSpec((1, DOUT, DOUT), lambda i: (0, 0, 0)),
            pl.BlockSpec((1, DOUT, DOUT), lambda i: (1, 0, 0)),
        ],
        out_specs=pl.BlockSpec((BN, DOUT), lambda i: (i, 0)),
        out_shape=jax.ShapeDtypeStruct((N, DOUT), _f32),
    )(agg1, agg1, g1f, g1f, dega, degb, b1r, w2r, w2r)


def _tc_final_body(agga_ref, aggb_ref, g2_ref, dega_ref, degb_ref,
                   b2_ref, bat_ref, out_ref, s_acc, c_acc):
    i = pl.program_id(0)

    @pl.when(i == 0)
    def _():
        s_acc[...] = jnp.zeros((G, DOUT), _f32)
        c_acc[...] = jnp.zeros((G, DOUT), _f32)

    deg = dega_ref[...] + degb_ref[...] + 1.0
    dinv = lax.rsqrt(deg)
    z2 = jnp.maximum(
        dinv * (agga_ref[...] + aggb_ref[...] + g2_ref[...]) + b2_ref[...],
        0.0)
    gids = lax.broadcasted_iota(jnp.int32, (BN, G), 1).astype(_f32)
    onehot = jnp.where(bat_ref[...] == gids, 1.0, 0.0)
    dn = (((0,), (0,)), ((), ()))
    s_acc[...] += lax.dot_general(onehot, z2, dn,
                                  preferred_element_type=_f32,
                                  precision=lax.Precision.HIGHEST)
    c_acc[...] += jnp.sum(onehot, axis=0)[:, None]

    @pl.when(i == pl.num_programs(0) - 1)
    def _():
        out_ref[...] = s_acc[...] / jnp.clip(c_acc[...], 1.0, None)


def _tc_final(agg2, g2, dega, degb, b2, batchf):
    return pl.pallas_call(
        _tc_final_body,
        grid=(NB,),
        in_specs=[
            pl.BlockSpec((BN, DOUT), lambda i: (i, 0)),
            pl.BlockSpec((BN, DOUT), lambda i: (i + NB, 0)),
            pl.BlockSpec((BN, DOUT), lambda i: (i, 0)),
            pl.BlockSpec((BN, 1), lambda i: (i, 0)),
            pl.BlockSpec((BN, 1), lambda i: (i, 0)),
            pl.BlockSpec((1, DOUT), lambda i: (0, 0)),
            pl.BlockSpec((BN, 1), lambda i: (i, 0)),
        ],
        out_specs=pl.BlockSpec((G, DOUT), lambda i: (0, 0)),
        out_shape=jax.ShapeDtypeStruct((G, DOUT), _f32),
        scratch_shapes=[pltpu.VMEM((G, DOUT), _f32),
                        pltpu.VMEM((G, DOUT), _f32)],
    )(agg2, agg2, g2, dega, degb, b2.reshape(1, DOUT), batchf)


def _agg_l1(g1f, src, dst):
    shp = (NS, NBLK, E // NS // K // NBLK, K)
    return _make_agg(DOUT, True)(g1f, src.reshape(shp), dst.reshape(shp))


def _agg_l2(g2, src, dst):
    shp = (NC * NS, NBLK, E // (NC * NS) // K // NBLK, K)
    return _make_agg(DOUT, False)(g2, src.reshape(shp), dst.reshape(shp))


def kernel(x, edge_index, batch, W1, b1, W2, b2):
    src = edge_index[0]
    dst = edge_index[1]
    batchf = batch.astype(_f32).reshape(N, 1)

    degp = _sc_degree(dst)
    degf = degp.reshape(2, HR * 128)
    dega = degf[0, :N].reshape(N, 1)
    degb = degf[1, :N].reshape(N, 1)
    g1 = _tc_prep(x, W1, dega, degb)
    g1f = g1.reshape(2 * N, DOUT)
    agg1 = _agg_l1(g1f, src, dst)
    g2 = _tc_mid(agg1, g1f, dega, degb, b1, W2)
    agg2 = _agg_l2(g2, src, dst)
    return _tc_final(agg2, g2, dega, degb, b2, batchf)


# speedup vs baseline: 21.2488x; 1.1597x over previous
import functools

import jax
import jax.numpy as jnp
from jax import lax
from jax.experimental import pallas as pl
from jax.experimental.pallas import tpu as pltpu
from jax.experimental.pallas import tpu_sc as plsc

N = 10000
E = 320000
DIN = 128
DOUT = 128
G = 16

NC = 2
NS = 16
LANES = 16
K = 80
ZR = 16
NBLK = 5

BN = 1000
NB = N // BN

@functools.lru_cache(maxsize=None)
def _mesh():
    return plsc.VectorSubcoreMesh(
        core_axis_name="c", subcore_axis_name="s",
        num_cores=NC, num_subcores=NS)

_f32 = jnp.float32


def _zero_shared(zbuf, shared, s, width):
    @pl.loop(0, ZR)
    def _(i):
        @pl.loop(0, width // LANES)
        def _(j):
            zbuf[i, pl.ds(j * LANES, LANES)] = jnp.zeros((LANES,), _f32)

    @pl.loop(0, 39)
    def _(d):
        pltpu.sync_copy(zbuf, shared.at[pl.ds(s * 624 + d * ZR, ZR)])

    @pl.when(s == NS - 1)
    def _():
        pltpu.sync_copy(zbuf.at[pl.ds(0, 16)], shared.at[pl.ds(9984, 16)])


HR = 80


def _sc_degree(dst):
    epw = E // (NC * NS)
    import dataclasses

    @functools.partial(
        pl.kernel,
        out_type=jax.ShapeDtypeStruct((NC * HR, 128), _f32),
        mesh=_mesh(),
        scratch_types=[
            pltpu.VMEM((epw,), jnp.int32),
            pltpu.VMEM((HR, 128), _f32),
            pltpu.VMEM((HR,), jnp.int32),
            pltpu.VMEM_SHARED((HR, 128), _f32),
        ],
        compiler_params=dataclasses.replace(pltpu.CompilerParams(),
                                            needs_layout_passes=False),
    )
    def deg_kernel(dst_hbm, out_hbm, idx_v, hist, iotab, shared):
        c = lax.axis_index("c")
        s = lax.axis_index("s")

        @pl.loop(0, HR)
        def _(i):
            @pl.loop(0, 8)
            def _(j):
                hist[i, pl.ds(j * LANES, LANES)] = jnp.zeros((LANES,), _f32)

        @pl.loop(0, HR // LANES)
        def _(i):
            iotab[pl.ds(i * LANES, LANES)] = (
                lax.iota(jnp.int32, LANES) + i * LANES)

        plsc.subcore_barrier()

        @pl.when(s == 0)
        def _():
            pltpu.sync_copy(hist, shared)

        pltpu.sync_copy(dst_hbm.at[c * NS + s], idx_v)
        plsc.subcore_barrier()

        ones = jnp.ones((LANES,), _f32)

        @pl.loop(0, epw // LANES)
        def _(t):
            iv = idx_v[pl.ds(t * LANES, LANES)]
            row = lax.shift_right_logical(iv, 7)
            lane = lax.bitwise_and(iv, 127)
            plsc.addupdate_scatter(hist, [row, lane], ones)

        plsc.subcore_barrier()
        pltpu.sync_copy(hist, shared.at[iotab], add=True)
        plsc.subcore_barrier()

        @pl.when(s == 0)
        def _():
            pltpu.sync_copy(shared, out_hbm.at[pl.ds(c * HR, HR)])

    return deg_kernel(dst.reshape(NC * NS, epw))


@functools.lru_cache(maxsize=None)
def _make_agg(width, col_split):
    epw = E // NS if col_split else E // (NC * NS)
    TB = epw // K // NBLK

    @functools.partial(
        pl.kernel,
        out_type=jax.ShapeDtypeStruct((NC * N, width), _f32),
        mesh=_mesh(),
        scratch_types=[
            pltpu.VMEM((TB, K), jnp.int32),
            pltpu.VMEM((TB, K), jnp.int32),
            pltpu.VMEM((K, width), _f32),
            pltpu.VMEM((K, width), _f32),
            pltpu.VMEM((K, width), _f32),
            pltpu.VMEM((ZR, width), _f32),
            pltpu.VMEM_SHARED((N, width), _f32),
            pltpu.SemaphoreType.DMA,
            pltpu.SemaphoreType.DMA,
            pltpu.SemaphoreType.DMA,
            pltpu.SemaphoreType.DMA,
            pltpu.SemaphoreType.DMA,
            pltpu.SemaphoreType.DMA,
        ],
    )
    def agg_kernel(g_hbm, src_hbm, dst_hbm, out_hbm,
                   sidx, didx, rows0, rows1, rows2, zbuf, shared,
                   gsem0, gsem1, gsem2, ssem0, ssem1, ssem2):
        c = lax.axis_index("c")
        s = lax.axis_index("s")
        _zero_shared(zbuf, shared, s, width)

        w = s if col_split else c * NS + s
        bufs = [(rows0, gsem0, ssem0), (rows1, gsem1, ssem1),
                (rows2, gsem2, ssem2)]
        NBUF = len(bufs)

        def gather(t, rows, sem):
            pltpu.async_copy(g_hbm.at[sidx.at[t]], rows, sem)

        def gather_wait(t, rows, sem):
            pltpu.make_async_copy(g_hbm.at[sidx.at[t]], rows, sem).wait()

        def scat(t, rows, sem):
            pltpu.async_copy(rows, shared.at[didx.at[t]], sem, add=True)

        def scat_wait(t, rows, sem):
            pltpu.make_async_copy(rows, shared.at[didx.at[t]], sem).wait()

        plsc.subcore_barrier()

        @pl.loop(0, NBLK)
        def _(blk):
            pltpu.sync_copy(src_hbm.at[w, blk], sidx)
            pltpu.sync_copy(dst_hbm.at[w, blk], didx)
            if col_split:
                off = c * N

                @pl.loop(0, TB)
                def _(t):
                    @pl.loop(0, K // LANES)
                    def _(j):
                        sidx[t, pl.ds(j * LANES, LANES)] = (
                            sidx[t, pl.ds(j * LANES, LANES)] + off)

            @pl.loop(0, TB // NBUF)
            def _(u):
                for i, (rows, gsem, ssem) in enumerate(bufs):
                    t = NBUF * u + i

                    @pl.when(u > 0)
                    def _():
                        scat_wait(t, rows, ssem)
                    gather(t, rows, gsem)
                for i, (rows, gsem, ssem) in enumerate(bufs):
                    t = NBUF * u + i
                    gather_wait(t, rows, gsem)
                    scat(t, rows, ssem)

            for i, (rows, _, ssem) in enumerate(bufs):
                scat_wait(i, rows, ssem)

            for i in range(TB % NBUF):
                t = TB - (TB % NBUF) + i
                rows, gsem, ssem = bufs[i]
                gather(t, rows, gsem)
                gather_wait(t, rows, gsem)
                scat(t, rows, ssem)
                scat_wait(t, rows, ssem)

        plsc.subcore_barrier()

        @pl.when(s == 0)
        def _():
            pltpu.sync_copy(shared, out_hbm.at[pl.ds(c * N, N)])

    return agg_kernel


def _tc_prep_body(x_ref, w_ref, dega_ref, degb_ref, out_ref):
    x = x_ref[...]
    xn = x / jnp.clip(jnp.sum(x, axis=-1, keepdims=True), 1.0, None)
    deg = dega_ref[...] + degb_ref[...] + 1.0
    dinv = lax.rsqrt(deg)
    h = lax.dot_general(xn, w_ref[...], (((1,), (0,)), ((), ())),
                        preferred_element_type=_f32,
                        precision=lax.Precision.HIGHEST)
    out_ref[0, :, :] = dinv * h


def _tc_prep(x, W1, dega, degb):
    return pl.pallas_call(
        _tc_prep_body,
        grid=(NB, 2),
        in_specs=[
            pl.BlockSpec((BN, DIN), lambda i, c: (i, 0)),
            pl.BlockSpec((DIN, DOUT), lambda i, c: (0, c)),
            pl.BlockSpec((BN, 1), lambda i, c: (i, 0)),
            pl.BlockSpec((BN, 1), lambda i, c: (i, 0)),
        ],
        out_specs=pl.BlockSpec((1, BN, DOUT), lambda i, c: (c, i, 0)),
        out_shape=jax.ShapeDtypeStruct((2, N, DOUT), _f32),
    )(x, W1, dega, degb)


def _tc_mid_body(agga_ref, aggb_ref, g1a_ref, g1b_ref, dega_ref, degb_ref,
                 b1_ref, w2a_ref, w2b_ref, out_ref):
    deg = dega_ref[...] + degb_ref[...] + 1.0
    dinv = lax.rsqrt(deg)
    b1 = b1_ref[...]
    z1a = jnp.maximum(dinv * (agga_ref[...] + g1a_ref[...]) + b1[0:1, :], 0.0)
    z1b = jnp.maximum(dinv * (aggb_ref[...] + g1b_ref[...]) + b1[1:2, :], 0.0)
    dn = (((1,), (0,)), ((), ()))
    h2 = (lax.dot_general(z1a, w2a_ref[0], dn, preferred_element_type=_f32,
                          precision=lax.Precision.HIGHEST)
          + lax.dot_general(z1b, w2b_ref[0], dn, preferred_element_type=_f32,
                            precision=lax.Precision.HIGHEST))
    out_ref[...] = dinv * h2


def _tc_mid(agg1, g1f, dega, degb, b1, W2):
    w2r = W2.reshape(2, DOUT, DOUT)
    b1r = b1.reshape(2, DOUT)
    return pl.pallas_call(
        _tc_mid_body,
        grid=(NB,),
        in_specs=[
            pl.BlockSpec((BN, DOUT), lambda i: (i, 0)),
            pl.BlockSpec((BN, DOUT), lambda i: (i + NB, 0)),
            pl.BlockSpec((BN, DOUT), lambda i: (i, 0)),
            pl.BlockSpec((BN, DOUT), lambda i: (i + NB, 0)),
            pl.BlockSpec((BN, 1), lambda i: (i, 0)),
            pl.BlockSpec((BN, 1), lambda i: (i, 0)),
            pl.BlockSpec((2, DOUT), lambda i: (0, 0)),
            pl.BlockSpec((1, DOUT, DOUT), lambda i: (0, 0, 0)),
            pl.BlockSpec((1, DOUT, DOUT), lambda i: (1, 0, 0)),
        ],
        out_specs=pl.BlockSpec((BN, DOUT), lambda i: (i, 0)),
        out_shape=jax.ShapeDtypeStruct((N, DOUT), _f32),
    )(agg1, agg1, g1f, g1f, dega, degb, b1r, w2r, w2r)


def _tc_final_body(agga_ref, aggb_ref, g2_ref, dega_ref, degb_ref,
                   b2_ref, bat_ref, out_ref, s_acc, c_acc):
    i = pl.program_id(0)

    @pl.when(i == 0)
    def _():
        s_acc[...] = jnp.zeros((G, DOUT), _f32)
        c_acc[...] = jnp.zeros((G, DOUT), _f32)

    deg = dega_ref[...] + degb_ref[...] + 1.0
    dinv = lax.rsqrt(deg)
    z2 = jnp.maximum(
        dinv * (agga_ref[...] + aggb_ref[...] + g2_ref[...]) + b2_ref[...],
        0.0)
    gids = lax.broadcasted_iota(jnp.int32, (BN, G), 1).astype(_f32)
    onehot = jnp.where(bat_ref[...] == gids, 1.0, 0.0)
    dn = (((0,), (0,)), ((), ()))
    s_acc[...] += lax.dot_general(onehot, z2, dn,
                                  preferred_element_type=_f32,
                                  precision=lax.Precision.HIGHEST)
    c_acc[...] += jnp.sum(onehot, axis=0)[:, None]

    @pl.when(i == pl.num_programs(0) - 1)
    def _():
        out_ref[...] = s_acc[...] / jnp.clip(c_acc[...], 1.0, None)


def _tc_final(agg2, g2, dega, degb, b2, batchf):
    return pl.pallas_call(
        _tc_final_body,
        grid=(NB,),
        in_specs=[
            pl.BlockSpec((BN, DOUT), lambda i: (i, 0)),
            pl.BlockSpec((BN, DOUT), lambda i: (i + NB, 0)),
            pl.BlockSpec((BN, DOUT), lambda i: (i, 0)),
            pl.BlockSpec((BN, 1), lambda i: (i, 0)),
            pl.BlockSpec((BN, 1), lambda i: (i, 0)),
            pl.BlockSpec((1, DOUT), lambda i: (0, 0)),
            pl.BlockSpec((BN, 1), lambda i: (i, 0)),
        ],
        out_specs=pl.BlockSpec((G, DOUT), lambda i: (0, 0)),
        out_shape=jax.ShapeDtypeStruct((G, DOUT), _f32),
        scratch_shapes=[pltpu.VMEM((G, DOUT), _f32),
                        pltpu.VMEM((G, DOUT), _f32)],
    )(agg2, agg2, g2, dega, degb, b2.reshape(1, DOUT), batchf)


def _agg_l1(g1f, src, dst):
    shp = (NS, NBLK, E // NS // K // NBLK, K)
    return _make_agg(DOUT, True)(g1f, src.reshape(shp), dst.reshape(shp))


def _agg_l2(g2, src, dst):
    shp = (NC * NS, NBLK, E // (NC * NS) // K // NBLK, K)
    return _make_agg(DOUT, False)(g2, src.reshape(shp), dst.reshape(shp))


def kernel(x, edge_index, batch, W1, b1, W2, b2):
    src = edge_index[0]
    dst = edge_index[1]
    batchf = batch.astype(_f32).reshape(N, 1)

    degp = _sc_degree(dst)
    degf = degp.reshape(2, HR * 128)
    dega = degf[0, :N].reshape(N, 1)
    degb = degf[1, :N].reshape(N, 1)
    g1 = _tc_prep(x, W1, dega, degb)
    g1f = g1.reshape(2 * N, DOUT)
    agg1 = _agg_l1(g1f, src, dst)
    g2 = _tc_mid(agg1, g1f, dega, degb, b1, W2)
    agg2 = _agg_l2(g2, src, dst)
    return _tc_final(agg2, g2, dega, degb, b2, batchf)


# speedup vs baseline: 21.8162x; 1.0267x over previous
import functools

import jax
import jax.numpy as jnp
from jax import lax
from jax.experimental import pallas as pl
from jax.experimental.pallas import tpu as pltpu
from jax.experimental.pallas import tpu_sc as plsc

N = 10000
E = 320000
DIN = 128
DOUT = 128
G = 16

NC = 2
NS = 16
LANES = 16
K = 80
ZR = 8


BN = 1000
NB = N // BN

@functools.lru_cache(maxsize=None)
def _mesh():
    return plsc.VectorSubcoreMesh(
        core_axis_name="c", subcore_axis_name="s",
        num_cores=NC, num_subcores=NS)

_f32 = jnp.float32


def _zero_shared(zbuf, shared, s, width):
    @pl.loop(0, ZR)
    def _(i):
        @pl.loop(0, width // LANES)
        def _(j):
            zbuf[i, pl.ds(j * LANES, LANES)] = jnp.zeros((LANES,), _f32)

    @pl.loop(0, 78)
    def _(d):
        pltpu.sync_copy(zbuf, shared.at[pl.ds(s * 624 + d * ZR, ZR)])

    @pl.when(s == NS - 1)
    def _():
        pltpu.sync_copy(zbuf, shared.at[pl.ds(9984, 8)])
        pltpu.sync_copy(zbuf, shared.at[pl.ds(9992, 8)])


HR = 80


def _sc_degree(dst):
    epw = E // (NC * NS)
    import dataclasses

    @functools.partial(
        pl.kernel,
        out_type=jax.ShapeDtypeStruct((NC * HR, 128), _f32),
        mesh=_mesh(),
        scratch_types=[
            pltpu.VMEM((epw,), jnp.int32),
            pltpu.VMEM((HR, 128), _f32),
            pltpu.VMEM((HR,), jnp.int32),
            pltpu.VMEM_SHARED((HR, 128), _f32),
        ],
        compiler_params=dataclasses.replace(pltpu.CompilerParams(),
                                            needs_layout_passes=False),
    )
    def deg_kernel(dst_hbm, out_hbm, idx_v, hist, iotab, shared):
        c = lax.axis_index("c")
        s = lax.axis_index("s")

        @pl.loop(0, HR)
        def _(i):
            @pl.loop(0, 8)
            def _(j):
                hist[i, pl.ds(j * LANES, LANES)] = jnp.zeros((LANES,), _f32)

        @pl.loop(0, HR // LANES)
        def _(i):
            iotab[pl.ds(i * LANES, LANES)] = (
                lax.iota(jnp.int32, LANES) + i * LANES)

        plsc.subcore_barrier()

        @pl.when(s == 0)
        def _():
            pltpu.sync_copy(hist, shared)

        pltpu.sync_copy(dst_hbm.at[c * NS + s], idx_v)
        plsc.subcore_barrier()

        ones = jnp.ones((LANES,), _f32)

        @pl.loop(0, epw // LANES)
        def _(t):
            iv = idx_v[pl.ds(t * LANES, LANES)]
            row = lax.shift_right_logical(iv, 7)
            lane = lax.bitwise_and(iv, 127)
            plsc.addupdate_scatter(hist, [row, lane], ones)

        plsc.subcore_barrier()
        pltpu.sync_copy(hist, shared.at[iotab], add=True)
        plsc.subcore_barrier()

        @pl.when(s == 0)
        def _():
            pltpu.sync_copy(shared, out_hbm.at[pl.ds(c * HR, HR)])

    return deg_kernel(dst.reshape(NC * NS, epw))


@functools.lru_cache(maxsize=None)
def _make_agg(width, col_split, nblk):
    epw = E // NS if col_split else E // (NC * NS)
    TB = epw // K // nblk

    @functools.partial(
        pl.kernel,
        out_type=jax.ShapeDtypeStruct((NC * N, width), _f32),
        mesh=_mesh(),
        scratch_types=[
            pltpu.VMEM((TB, K), jnp.int32),
            pltpu.VMEM((TB, K), jnp.int32),
            pltpu.VMEM((K, width), _f32),
            pltpu.VMEM((K, width), _f32),
            pltpu.VMEM((K, width), _f32),
            pltpu.VMEM((K, width), _f32),
            pltpu.VMEM((ZR, width), _f32),
            pltpu.VMEM_SHARED((N, width), _f32),
            pltpu.SemaphoreType.DMA,
            pltpu.SemaphoreType.DMA,
            pltpu.SemaphoreType.DMA,
            pltpu.SemaphoreType.DMA,
            pltpu.SemaphoreType.DMA,
            pltpu.SemaphoreType.DMA,
            pltpu.SemaphoreType.DMA,
            pltpu.SemaphoreType.DMA,
        ],
    )
    def agg_kernel(g_hbm, src_hbm, dst_hbm, out_hbm,
                   sidx, didx, rows0, rows1, rows2, rows3, zbuf, shared,
                   gsem0, gsem1, gsem2, gsem3, ssem0, ssem1, ssem2, ssem3):
        c = lax.axis_index("c")
        s = lax.axis_index("s")
        _zero_shared(zbuf, shared, s, width)

        w = s if col_split else c * NS + s
        bufs = [(rows0, gsem0, ssem0), (rows1, gsem1, ssem1),
                (rows2, gsem2, ssem2), (rows3, gsem3, ssem3)]
        NBUF = len(bufs)

        def gather(t, rows, sem):
            pltpu.async_copy(g_hbm.at[sidx.at[t]], rows, sem)

        def gather_wait(t, rows, sem):
            pltpu.make_async_copy(g_hbm.at[sidx.at[t]], rows, sem).wait()

        def scat(t, rows, sem):
            pltpu.async_copy(rows, shared.at[didx.at[t]], sem, add=True)

        def scat_wait(t, rows, sem):
            pltpu.make_async_copy(rows, shared.at[didx.at[t]], sem).wait()

        plsc.subcore_barrier()

        @pl.loop(0, nblk)
        def _(blk):
            pltpu.sync_copy(src_hbm.at[w, blk], sidx)
            pltpu.sync_copy(dst_hbm.at[w, blk], didx)
            if col_split:
                off = c * N

                @pl.loop(0, TB)
                def _(t):
                    @pl.loop(0, K // LANES)
                    def _(j):
                        sidx[t, pl.ds(j * LANES, LANES)] = (
                            sidx[t, pl.ds(j * LANES, LANES)] + off)

            @pl.loop(0, TB // NBUF)
            def _(u):
                for i, (rows, gsem, ssem) in enumerate(bufs):
                    t = NBUF * u + i

                    @pl.when(u > 0)
                    def _():
                        scat_wait(t, rows, ssem)
                    gather(t, rows, gsem)
                for i, (rows, gsem, ssem) in enumerate(bufs):
                    t = NBUF * u + i
                    gather_wait(t, rows, gsem)
                    scat(t, rows, ssem)

            for i, (rows, _, ssem) in enumerate(bufs):
                scat_wait(i, rows, ssem)

            for i in range(TB % NBUF):
                t = TB - (TB % NBUF) + i
                rows, gsem, ssem = bufs[i]
                gather(t, rows, gsem)
                gather_wait(t, rows, gsem)
                scat(t, rows, ssem)
                scat_wait(t, rows, ssem)

        plsc.subcore_barrier()

        @pl.when(s == 0)
        def _():
            pltpu.sync_copy(shared, out_hbm.at[pl.ds(c * N, N)])

    return agg_kernel


def _tc_prep_body(x_ref, w_ref, dega_ref, degb_ref, out_ref):
    x = x_ref[...]
    xn = x / jnp.clip(jnp.sum(x, axis=-1, keepdims=True), 1.0, None)
    deg = dega_ref[...] + degb_ref[...] + 1.0
    dinv = lax.rsqrt(deg)
    h = lax.dot_general(xn, w_ref[...], (((1,), (0,)), ((), ())),
                        preferred_element_type=_f32,
                        precision=lax.Precision.HIGHEST)
    out_ref[0, :, :] = dinv * h


def _tc_prep(x, W1, dega, degb):
    return pl.pallas_call(
        _tc_prep_body,
        grid=(NB, 2),
        in_specs=[
            pl.BlockSpec((BN, DIN), lambda i, c: (i, 0)),
            pl.BlockSpec((DIN, DOUT), lambda i, c: (0, c)),
            pl.BlockSpec((BN, 1), lambda i, c: (i, 0)),
            pl.BlockSpec((BN, 1), lambda i, c: (i, 0)),
        ],
        out_specs=pl.BlockSpec((1, BN, DOUT), lambda i, c: (c, i, 0)),
        out_shape=jax.ShapeDtypeStruct((2, N, DOUT), _f32),
    )(x, W1, dega, degb)


def _tc_mid_body(agga_ref, aggb_ref, g1a_ref, g1b_ref, dega_ref, degb_ref,
                 b1_ref, w2a_ref, w2b_ref, out_ref):
    deg = dega_ref[...] + degb_ref[...] + 1.0
    dinv = lax.rsqrt(deg)
    b1 = b1_ref[...]
    z1a = jnp.maximum(dinv * (agga_ref[...] + g1a_ref[...]) + b1[0:1, :], 0.0)
    z1b = jnp.maximum(dinv * (aggb_ref[...] + g1b_ref[...]) + b1[1:2, :], 0.0)
    dn = (((1,), (0,)), ((), ()))
    h2 = (lax.dot_general(z1a, w2a_ref[0], dn, preferred_element_type=_f32,
                          precision=lax.Precision.HIGHEST)
          + lax.dot_general(z1b, w2b_ref[0], dn, preferred_element_type=_f32,
                            precision=lax.Precision.HIGHEST))
    out_ref[...] = dinv * h2


def _tc_mid(agg1, g1f, dega, degb, b1, W2):
    w2r = W2.reshape(2, DOUT, DOUT)
    b1r = b1.reshape(2, DOUT)
    return pl.pallas_call(
        _tc_mid_body,
        grid=(NB,),
        in_specs=[
            pl.BlockSpec((BN, DOUT), lambda i: (i, 0)),
            pl.BlockSpec((BN, DOUT), lambda i: (i + NB, 0)),
            pl.BlockSpec((BN, DOUT), lambda i: (i, 0)),
            pl.BlockSpec((BN, DOUT), lambda i: (i + NB, 0)),
            pl.BlockSpec((BN, 1), lambda i: (i, 0)),
            pl.BlockSpec((BN, 1), lambda i: (i, 0)),
            pl.BlockSpec((2, DOUT), lambda i: (0, 0)),
            pl.BlockSpec((1, DOUT, DOUT), lambda i: (0, 0, 0)),
            pl.BlockSpec((1, DOUT, DOUT), lambda i: (1, 0, 0)),
        ],
        out_specs=pl.BlockSpec((BN, DOUT), lambda i: (i, 0)),
        out_shape=jax.ShapeDtypeStruct((N, DOUT), _f32),
    )(agg1, agg1, g1f, g1f, dega, degb, b1r, w2r, w2r)


def _tc_final_body(agga_ref, aggb_ref, g2_ref, dega_ref, degb_ref,
                   b2_ref, bat_ref, out_ref, s_acc, c_acc):
    i = pl.program_id(0)

    @pl.when(i == 0)
    def _():
        s_acc[...] = jnp.zeros((G, DOUT), _f32)
        c_acc[...] = jnp.zeros((G, DOUT), _f32)

    deg = dega_ref[...] + degb_ref[...] + 1.0
    dinv = lax.rsqrt(deg)
    z2 = jnp.maximum(
        dinv * (agga_ref[...] + aggb_ref[...] + g2_ref[...]) + b2_ref[...],
        0.0)
    gids = lax.broadcasted_iota(jnp.int32, (BN, G), 1).astype(_f32)
    onehot = jnp.where(bat_ref[...] == gids, 1.0, 0.0)
    dn = (((0,), (0,)), ((), ()))
    s_acc[...] += lax.dot_general(onehot, z2, dn,
                                  preferred_element_type=_f32,
                                  precision=lax.Precision.HIGHEST)
    c_acc[...] += jnp.sum(onehot, axis=0)[:, None]

    @pl.when(i == pl.num_programs(0) - 1)
    def _():
        out_ref[...] = s_acc[...] / jnp.clip(c_acc[...], 1.0, None)


def _tc_final(agg2, g2, dega, degb, b2, batchf):
    return pl.pallas_call(
        _tc_final_body,
        grid=(NB,),
        in_specs=[
            pl.BlockSpec((BN, DOUT), lambda i: (i, 0)),
            pl.BlockSpec((BN, DOUT), lambda i: (i + NB, 0)),
            pl.BlockSpec((BN, DOUT), lambda i: (i, 0)),
            pl.BlockSpec((BN, 1), lambda i: (i, 0)),
            pl.BlockSpec((BN, 1), lambda i: (i, 0)),
            pl.BlockSpec((1, DOUT), lambda i: (0, 0)),
            pl.BlockSpec((BN, 1), lambda i: (i, 0)),
        ],
        out_specs=pl.BlockSpec((G, DOUT), lambda i: (0, 0)),
        out_shape=jax.ShapeDtypeStruct((G, DOUT), _f32),
        scratch_shapes=[pltpu.VMEM((G, DOUT), _f32),
                        pltpu.VMEM((G, DOUT), _f32)],
    )(agg2, agg2, g2, dega, degb, b2.reshape(1, DOUT), batchf)


def _agg_l1(g1f, src, dst):
    shp = (NS, 10, E // NS // K // 10, K)
    return _make_agg(DOUT, True, 10)(g1f, src.reshape(shp), dst.reshape(shp))


def _agg_l2(g2, src, dst):
    shp = (NC * NS, 5, E // (NC * NS) // K // 5, K)
    return _make_agg(DOUT, False, 5)(g2, src.reshape(shp), dst.reshape(shp))


def kernel(x, edge_index, batch, W1, b1, W2, b2):
    src = edge_index[0]
    dst = edge_index[1]
    batchf = batch.astype(_f32).reshape(N, 1)

    degp = _sc_degree(dst)
    degf = degp.reshape(2, HR * 128)
    dega = degf[0, :N].reshape(N, 1)
    degb = degf[1, :N].reshape(N, 1)
    g1 = _tc_prep(x, W1, dega, degb)
    g1f = g1.reshape(2 * N, DOUT)
    agg1 = _agg_l1(g1f, src, dst)
    g2 = _tc_mid(agg1, g1f, dega, degb, b1, W2)
    agg2 = _agg_l2(g2, src, dst)
    return _tc_final(agg2, g2, dega, degb, b2, batchf)
